# baseline (speedup 1.0000x reference)
import functools

import jax
import jax.numpy as jnp
from jax import lax
from jax.experimental import pallas as pl
from jax.experimental.pallas import tpu as pltpu
from jax.experimental.pallas import tpu_sc as plsc

NC = 2
NS = 16
NW = NC * NS
L = 16
CH = 40
NB = 4
PF = 2


def _lane_broadcast(v, l):
    return lax.gather(
        v, jnp.full((L, 1), l, jnp.int32),
        dimension_numbers=lax.GatherDimensionNumbers(
            offset_dims=(), collapsed_slice_dims=(0,), start_index_map=(0,)),
        slice_sizes=(1,),
        mode=lax.GatherScatterMode.PROMISE_IN_BOUNDS)


def kernel(x, positions, embedding_weight):
    B, S, D = x.shape
    V = embedding_weight.shape[0]
    tabf = embedding_weight.reshape(V * D)
    pf = positions.reshape(B * S)
    BW = B // NW
    CPB = S // CH
    NCH = BW * CPB

    mesh = plsc.VectorSubcoreMesh(core_axis_name="c", subcore_axis_name="s")

    @functools.partial(
        pl.kernel,
        out_type=jax.ShapeDtypeStruct((B, S, D), jnp.float32),
        mesh=mesh,
        compiler_params=pltpu.CompilerParams(
            needs_layout_passes=False, use_tc_tiling_on_sc=False),
        scratch_types=(
            [pltpu.VMEM((V * D,), jnp.float32)]
            + [pltpu.VMEM((CH, D), jnp.float32) for _ in range(NB)]
            + [pltpu.VMEM((CH,), jnp.int32) for _ in range(NB)]
            + [pltpu.SemaphoreType.DMA for _ in range(3 * NB)]
        ),
    )
    def sc_kernel(x_hbm, pos_hbm, tab_hbm, out_hbm, tab_v, *scratch):
        xbufs = scratch[0:NB]
        idxs = scratch[NB:2 * NB]
        sems = scratch[2 * NB:]
        sx = sems[0:NB]
        sp = sems[NB:2 * NB]
        so = sems[2 * NB:3 * NB]

        iota16 = lax.iota(jnp.int32, L)
        cid = lax.axis_index("c")
        sid = lax.axis_index("s")
        wid = sid * NC + cid
        b0 = wid * BW
        row0 = b0 * S
        pltpu.sync_copy(tab_hbm, tab_v)

        def slab_view(hbm, k):
            return hbm.at[b0 + k // CPB, pl.ds((k % CPB) * CH, CH)]

        def issue_in(k, b):
            pltpu.async_copy(pos_hbm.at[pl.ds(row0 + k * CH, CH)], idxs[b],
                             sp[b])
            pltpu.async_copy(slab_view(x_hbm, k), xbufs[b], sx[b])

        for k0 in range(PF):
            issue_in(k0, k0 % NB)

        def do_rows(b, rows64, base, l0, l1):
            for l in range(l0, l1):
                rsplat = _lane_broadcast(rows64, l)
                for cb in range(D // L):
                    vals = plsc.load_gather(
                        tab_v, [rsplat + (iota16 + cb * L)])
                    plsc.addupdate(
                        xbufs[b].at[base + l, pl.ds(cb * L, L)], vals)

        @pl.loop(0, NCH, step=NB)
        def _(r):
            for j in range(NB):
                k = r + j
                b = j
                pltpu.make_async_copy(
                    pos_hbm.at[pl.ds(row0 + k * CH, CH)], idxs[b],
                    sp[b]).wait()
                pltpu.make_async_copy(
                    slab_view(x_hbm, k), xbufs[b], sx[b]).wait()

                for g in (0, L):
                    do_rows(b, idxs[b][pl.ds(g, L)] * D, g, 0, L)
                do_rows(b, idxs[b][pl.ds(CH - L, L)] * D, CH - L, L // 2, L)

                pltpu.async_copy(xbufs[b], slab_view(out_hbm, k), so[b])

                bn = (j + PF) % NB

                @pl.when(k >= PF)
                def _():
                    pltpu.make_async_copy(
                        xbufs[bn], slab_view(out_hbm, 0), so[bn]).wait()

                @pl.when(k + PF < NCH)
                def _():
                    issue_in(k + PF, bn)

        for k0 in range(NCH - PF, NCH):
            b = k0 % NB
            pltpu.make_async_copy(
                xbufs[b], slab_view(out_hbm, 0), so[b]).wait()

    return sc_kernel(x, pf, tabf)


# speedup vs baseline: 1.2504x; 1.2504x over previous
import functools

import jax
import jax.numpy as jnp
from jax import lax
from jax.experimental import pallas as pl
from jax.experimental.pallas import tpu as pltpu
from jax.experimental.pallas import tpu_sc as plsc

NC = 2
NS = 16
NW = NC * NS
L = 16
C = 256
NB = 4
PF = 2


def kernel(x, positions, embedding_weight):
    B, S, D = x.shape
    V = embedding_weight.shape[0]
    N = B * S
    xf = x.reshape(N * D)
    pf = positions.reshape(N)
    tabf = embedding_weight.reshape(V * D)
    R = N // NW
    NCHUNK = R // C

    mesh = plsc.VectorSubcoreMesh(core_axis_name="c", subcore_axis_name="s")

    @functools.partial(
        pl.kernel,
        out_type=jax.ShapeDtypeStruct((N * D,), jnp.float32),
        mesh=mesh,
        compiler_params=pltpu.CompilerParams(needs_layout_passes=False),
        scratch_types=(
            [pltpu.VMEM((V * D,), jnp.float32)]
            + [pltpu.VMEM((C * D,), jnp.float32) for _ in range(NB)]
            + [pltpu.VMEM((C,), jnp.int32) for _ in range(NB)]
            + [pltpu.SemaphoreType.DMA for _ in range(3 * NB)]
        ),
    )
    def sc_kernel(x_hbm, pos_hbm, tab_hbm, out_hbm, tab_v, *scratch):
        xbufs = scratch[0:NB]
        idxs = scratch[NB:2 * NB]
        sems = scratch[2 * NB:]
        sx = sems[0:NB]
        sp = sems[NB:2 * NB]
        so = sems[2 * NB:3 * NB]

        iota16 = lax.iota(jnp.int32, L)
        cid = lax.axis_index("c")
        sid = lax.axis_index("s")
        wid = sid * NC + cid
        row0 = wid * R
        pltpu.sync_copy(tab_hbm, tab_v)

        def issue_in(k, b):
            rbase = row0 + k * C
            pltpu.async_copy(pos_hbm.at[pl.ds(rbase, C)], idxs[b], sp[b])
            pltpu.async_copy(x_hbm.at[pl.ds(rbase * D, C * D)], xbufs[b], sx[b])

        for k0 in range(PF):
            issue_in(k0, k0 % NB)

        @pl.loop(0, NCHUNK, step=NB)
        def _(r):
            for j in range(NB):
                k = r + j
                b = j
                rbase = row0 + k * C
                pltpu.make_async_copy(
                    pos_hbm.at[pl.ds(rbase, C)], idxs[b], sp[b]).wait()
                pltpu.make_async_copy(
                    x_hbm.at[pl.ds(rbase * D, C * D)], xbufs[b], sx[b]).wait()

                @pl.loop(0, C, step=L)
                def _(g):
                    rows = idxs[b][pl.ds(g, L)] * D
                    gD = g * D
                    for l in range(L):
                        rsplat = lax.gather(
                            rows, jnp.full((L, 1), l, jnp.int32),
                            dimension_numbers=lax.GatherDimensionNumbers(
                                offset_dims=(), collapsed_slice_dims=(0,),
                                start_index_map=(0,)),
                            slice_sizes=(1,),
                            mode=lax.GatherScatterMode.PROMISE_IN_BOUNDS)
                        for cb in range(D // L):
                            vals = plsc.load_gather(
                                tab_v, [rsplat + (iota16 + cb * L)])
                            plsc.addupdate(
                                xbufs[b].at[pl.ds(gD + l * D + cb * L, L)],
                                vals)

                pltpu.async_copy(
                    xbufs[b], out_hbm.at[pl.ds(rbase * D, C * D)], so[b])

                bn = (j + PF) % NB

                @pl.when(k >= PF)
                def _():
                    pltpu.make_async_copy(
                        xbufs[bn], out_hbm.at[pl.ds(0, C * D)], so[bn]).wait()

                @pl.when(k + PF < NCHUNK)
                def _():
                    issue_in(k + PF, bn)

        for k0 in range(NCHUNK - PF, NCHUNK):
            b = k0 % NB
            pltpu.make_async_copy(
                xbufs[b], out_hbm.at[pl.ds(0, C * D)], so[b]).wait()

    out = sc_kernel(xf, pf, tabf)
    return out.reshape(B, S, D)
